# baseline (speedup 1.0000x reference)
import functools

import jax
import jax.numpy as jnp
import numpy as np
from jax import lax
from jax.experimental import pallas as pl
from jax.experimental.pallas import tpu as pltpu
from jax.experimental.pallas import tpu_sc as plsc

_H = 8
_N = 38416
_NV = _N // 16
_NW = 32
_VPW = 76
_CHUNK = _VPW * 16


def _bb_pos_flat(qq, kk):
    strides = int(np.ceil(np.sqrt(float(kk) / float(qq))))
    q_h = int(np.sqrt(float(qq)))
    k_h = int(np.sqrt(float(kk)))
    x1, y1 = np.meshgrid(np.arange(q_h), np.arange(q_h))
    x2, y2 = np.meshgrid(np.arange(k_h), np.arange(k_h))
    aa = np.concatenate([x1.reshape(-1, 1), y1.reshape(-1, 1)], axis=-1)
    b2 = np.concatenate([x2.reshape(-1, 1), y2.reshape(-1, 1)], axis=-1)
    cc = np.abs(b2[None, :, :] - aa[:, None, :] * strides)
    pos = cc[:, :, 0] + cc[:, :, 1] * k_h
    return pos.reshape(-1).astype(np.int32)


def _sc_gather_body(bb_hbm, idx_hbm, out_hbm, idx_v, bb_v, res_v):
    wid = lax.axis_index("s") * 2 + lax.axis_index("c")
    base = jnp.minimum(wid * _CHUNK, _N - _CHUNK)
    pltpu.sync_copy(bb_hbm, bb_v)
    pltpu.sync_copy(idx_hbm.at[pl.ds(base, _CHUNK)], idx_v)

    def j_body(j, carry):
        idx8 = idx_v[pl.ds(j * 16, 16)] * 8
        for h in range(_H):
            res_v[pl.ds(h * _CHUNK + j * 16, 16)] = plsc.load_gather(bb_v, [idx8 + h])
        return carry

    lax.fori_loop(0, _VPW, j_body, 0)
    for h in range(_H):
        pltpu.sync_copy(res_v.at[pl.ds(h * _CHUNK, _CHUNK)],
                        out_hbm.at[pl.ds(h * _N + base, _CHUNK)])


def _sc_gather(bb, idx_flat):
    mesh = plsc.VectorSubcoreMesh(core_axis_name="c", subcore_axis_name="s")
    return pl.kernel(
        _sc_gather_body,
        mesh=mesh,
        compiler_params=pltpu.CompilerParams(needs_layout_passes=False),
        out_type=jax.ShapeDtypeStruct((_H * _N,), jnp.float32),
        scratch_types=[
            pltpu.VMEM((_CHUNK,), jnp.int32),
            pltpu.VMEM((196 * _H,), jnp.float32),
            pltpu.VMEM((_H * _CHUNK,), jnp.float32),
        ],
    )(bb, idx_flat)


def _add_body(x_ref, pb_ref, o_ref):
    o_ref[...] = x_ref[...] + pb_ref[...]


def _tc_add(x4, pb4):
    B, H, QQ, KK = x4.shape
    bblk = 4
    return pl.pallas_call(
        _add_body,
        grid=(B // bblk,),
        in_specs=[
            pl.BlockSpec((bblk, H, QQ, KK), lambda b: (b, 0, 0, 0)),
            pl.BlockSpec((1, H, QQ, KK), lambda b: (0, 0, 0, 0)),
        ],
        out_specs=pl.BlockSpec((bblk, H, QQ, KK), lambda b: (b, 0, 0, 0)),
        out_shape=jax.ShapeDtypeStruct((B, H, QQ, KK), jnp.float32),
    )(x4, pb4)


@jax.jit
def kernel(inputs, bb):
    B, H, QQ, KK = inputs.shape
    idx_flat = jnp.asarray(_bb_pos_flat(QQ, KK))
    pb = _sc_gather(bb.reshape(-1), idx_flat)
    return _tc_add(inputs, pb.reshape(1, H, QQ, KK))


# speedup vs baseline: 3.0049x; 3.0049x over previous
import functools

import jax
import jax.numpy as jnp
import numpy as np
from jax import lax
from jax.experimental import pallas as pl
from jax.experimental.pallas import tpu as pltpu
from jax.experimental.pallas import tpu_sc as plsc

_H = 8
_N = 38416
_NV = _N // 16
_NW = 32
_VPW = 76
_CHUNK = _VPW * 16


def _bb_pos_flat(qq, kk):
    strides = int(np.ceil(np.sqrt(float(kk) / float(qq))))
    q_h = int(np.sqrt(float(qq)))
    k_h = int(np.sqrt(float(kk)))
    x1, y1 = np.meshgrid(np.arange(q_h), np.arange(q_h))
    x2, y2 = np.meshgrid(np.arange(k_h), np.arange(k_h))
    aa = np.concatenate([x1.reshape(-1, 1), y1.reshape(-1, 1)], axis=-1)
    b2 = np.concatenate([x2.reshape(-1, 1), y2.reshape(-1, 1)], axis=-1)
    cc = np.abs(b2[None, :, :] - aa[:, None, :] * strides)
    pos = cc[:, :, 0] + cc[:, :, 1] * k_h
    return pos.reshape(-1).astype(np.int32)


def _sc_gather_body(bb_hbm, idx_hbm, out_hbm, idx_v, bb_v, res_v):
    wid = lax.axis_index("s") * 2 + lax.axis_index("c")
    base = jnp.minimum(wid * _CHUNK, _N - _CHUNK)
    pltpu.sync_copy(bb_hbm, bb_v)
    pltpu.sync_copy(idx_hbm.at[pl.ds(base, _CHUNK)], idx_v)

    def j_body(j, carry):
        idx8 = idx_v[pl.ds(j * 16, 16)] * 8
        for h in range(_H):
            res_v[pl.ds(h * _CHUNK + j * 16, 16)] = plsc.load_gather(bb_v, [idx8 + h])
        return carry

    lax.fori_loop(0, _VPW, j_body, 0)
    for h in range(_H):
        pltpu.sync_copy(res_v.at[pl.ds(h * _CHUNK, _CHUNK)],
                        out_hbm.at[pl.ds(h * _N + base, _CHUNK)])


def _sc_gather(bb, idx_flat):
    mesh = plsc.VectorSubcoreMesh(core_axis_name="c", subcore_axis_name="s")
    return pl.kernel(
        _sc_gather_body,
        mesh=mesh,
        compiler_params=pltpu.CompilerParams(needs_layout_passes=False),
        out_type=jax.ShapeDtypeStruct((_H * _N,), jnp.float32),
        scratch_types=[
            pltpu.VMEM((_CHUNK,), jnp.int32),
            pltpu.VMEM((196 * _H,), jnp.float32),
            pltpu.VMEM((_H * _CHUNK,), jnp.float32),
        ],
    )(bb, idx_flat)


def _add_body(x_ref, pb_ref, o_ref):
    def g_body(g, carry):
        for j in range(16):
            col = pb_ref[g][:, j:j + 1]
            o_ref[g, j] = x_ref[g, j] + jnp.broadcast_to(col, (8, 128))
        return carry

    lax.fori_loop(0, _GBLK, g_body, 0)


_G = 2401
_GBLK = 49


def _tc_add(xt, pbt):
    return pl.pallas_call(
        _add_body,
        grid=(_G // _GBLK,),
        in_specs=[
            pl.BlockSpec((_GBLK, 16, 8, 128), lambda i: (i, 0, 0, 0)),
            pl.BlockSpec((_GBLK, 8, 16), lambda i: (i, 0, 0)),
        ],
        out_specs=pl.BlockSpec((_GBLK, 16, 8, 128), lambda i: (i, 0, 0, 0)),
        out_shape=jax.ShapeDtypeStruct((_G, 16, 8, 128), jnp.float32),
    )(xt, pbt)


@jax.jit
def kernel(inputs, bb):
    B, H, QQ, KK = inputs.shape
    idx_flat = jnp.asarray(_bb_pos_flat(QQ, KK))
    pb = _sc_gather(bb.reshape(-1), idx_flat)
    xt = jnp.transpose(inputs, (2, 3, 1, 0)).reshape(_G, 16, H, B)
    pbt = jnp.transpose(pb.reshape(H, _G, 16), (1, 0, 2))
    ot = _tc_add(xt, pbt)
    return jnp.transpose(ot.reshape(QQ, KK, H, B), (3, 2, 0, 1))


# speedup vs baseline: 5.4509x; 1.8140x over previous
import functools

import jax
import jax.numpy as jnp
import numpy as np
from jax import lax
from jax.experimental import pallas as pl
from jax.experimental.pallas import tpu as pltpu
from jax.experimental.pallas import tpu_sc as plsc

_H = 8
_N = 38416
_NV = _N // 16
_NW = 32
_VPW = 76
_CHUNK = _VPW * 16


def _bb_pos_flat(qq, kk):
    strides = int(np.ceil(np.sqrt(float(kk) / float(qq))))
    q_h = int(np.sqrt(float(qq)))
    k_h = int(np.sqrt(float(kk)))
    x1, y1 = np.meshgrid(np.arange(q_h), np.arange(q_h))
    x2, y2 = np.meshgrid(np.arange(k_h), np.arange(k_h))
    aa = np.concatenate([x1.reshape(-1, 1), y1.reshape(-1, 1)], axis=-1)
    b2 = np.concatenate([x2.reshape(-1, 1), y2.reshape(-1, 1)], axis=-1)
    cc = np.abs(b2[None, :, :] - aa[:, None, :] * strides)
    pos = cc[:, :, 0] + cc[:, :, 1] * k_h
    return pos.reshape(-1).astype(np.int32)


def _sc_gather_body(bb_hbm, idx_hbm, out_hbm, idx_v, bb_v, res_v):
    wid = lax.axis_index("s") * 2 + lax.axis_index("c")
    base = jnp.minimum(wid * _CHUNK, _N - _CHUNK)
    pltpu.sync_copy(bb_hbm, bb_v)
    pltpu.sync_copy(idx_hbm.at[pl.ds(base, _CHUNK)], idx_v)

    def j_body(j, carry):
        idx8 = idx_v[pl.ds(j * 16, 16)] * 8
        for h in range(_H):
            res_v[pl.ds(h * _CHUNK + j * 16, 16)] = plsc.load_gather(bb_v, [idx8 + h])
        return carry

    lax.fori_loop(0, _VPW, j_body, 0)
    for h in range(_H):
        pltpu.sync_copy(res_v.at[pl.ds(h * _CHUNK, _CHUNK)],
                        out_hbm.at[pl.ds(h * _N + base, _CHUNK)])


def _sc_gather(bb, idx_flat):
    mesh = plsc.VectorSubcoreMesh(core_axis_name="c", subcore_axis_name="s")
    return pl.kernel(
        _sc_gather_body,
        mesh=mesh,
        compiler_params=pltpu.CompilerParams(needs_layout_passes=False),
        out_type=jax.ShapeDtypeStruct((_H * _N,), jnp.float32),
        scratch_types=[
            pltpu.VMEM((_CHUNK,), jnp.int32),
            pltpu.VMEM((196 * _H,), jnp.float32),
            pltpu.VMEM((_H * _CHUNK,), jnp.float32),
        ],
    )(bb, idx_flat)


def _add_body(x_ref, pb_ref, o_ref):
    row = jax.lax.broadcasted_iota(jnp.int32, (16, 2048), 0)
    col = jax.lax.broadcasted_iota(jnp.int32, (16, 2048), 1)
    rep = (row == col // 128).astype(jnp.float32)
    for g in range(_GBLK):
        t = jax.lax.dot_general(pb_ref[g], rep, (((1,), (0,)), ((), ())),
                                preferred_element_type=jnp.float32)
        for j in range(16):
            o_ref[g, j] = x_ref[g, j] + t[:, j * 128:(j + 1) * 128]


_G = 2401
_GBLK = 49


def _tc_add(xt, pbt):
    return pl.pallas_call(
        _add_body,
        grid=(_G // _GBLK,),
        in_specs=[
            pl.BlockSpec((_GBLK, 16, 8, 128), lambda i: (i, 0, 0, 0)),
            pl.BlockSpec((_GBLK, 8, 16), lambda i: (i, 0, 0)),
        ],
        out_specs=pl.BlockSpec((_GBLK, 16, 8, 128), lambda i: (i, 0, 0, 0)),
        out_shape=jax.ShapeDtypeStruct((_G, 16, 8, 128), jnp.float32),
    )(xt, pbt)


@jax.jit
def kernel(inputs, bb):
    B, H, QQ, KK = inputs.shape
    idx_flat = jnp.asarray(_bb_pos_flat(QQ, KK))
    pb = _sc_gather(bb.reshape(-1), idx_flat)
    xt = jnp.transpose(inputs, (2, 3, 1, 0)).reshape(_G, 16, H, B)
    pbt = jnp.transpose(pb.reshape(H, _G, 16), (1, 0, 2))
    ot = _tc_add(xt, pbt)
    return jnp.transpose(ot.reshape(QQ, KK, H, B), (3, 2, 0, 1))


# speedup vs baseline: 5.6544x; 1.0373x over previous
import functools

import jax
import jax.numpy as jnp
import numpy as np
from jax import lax
from jax.experimental import pallas as pl
from jax.experimental.pallas import tpu as pltpu
from jax.experimental.pallas import tpu_sc as plsc

_H = 8
_N = 38416
_NV = _N // 16
_NW = 32
_VPW = 76
_CHUNK = _VPW * 16


def _bb_pos_flat(qq, kk):
    strides = int(np.ceil(np.sqrt(float(kk) / float(qq))))
    q_h = int(np.sqrt(float(qq)))
    k_h = int(np.sqrt(float(kk)))
    x1, y1 = np.meshgrid(np.arange(q_h), np.arange(q_h))
    x2, y2 = np.meshgrid(np.arange(k_h), np.arange(k_h))
    aa = np.concatenate([x1.reshape(-1, 1), y1.reshape(-1, 1)], axis=-1)
    b2 = np.concatenate([x2.reshape(-1, 1), y2.reshape(-1, 1)], axis=-1)
    cc = np.abs(b2[None, :, :] - aa[:, None, :] * strides)
    pos = cc[:, :, 0] + cc[:, :, 1] * k_h
    return pos.reshape(-1).astype(np.int32)


def _sc_gather_body(bb_hbm, idx_hbm, out_hbm, idx_v, bb_v, res_v):
    wid = lax.axis_index("s") * 2 + lax.axis_index("c")
    base = jnp.minimum(wid * _CHUNK, _N - _CHUNK)
    pltpu.sync_copy(bb_hbm, bb_v)
    pltpu.sync_copy(idx_hbm.at[pl.ds(base, _CHUNK)], idx_v)

    def j_body(j, carry):
        idx8 = idx_v[pl.ds(j * 16, 16)] * 8
        for h in range(_H):
            res_v[pl.ds(h * _CHUNK + j * 16, 16)] = plsc.load_gather(bb_v, [idx8 + h])
        return carry

    lax.fori_loop(0, _VPW, j_body, 0)
    for h in range(_H):
        pltpu.sync_copy(res_v.at[pl.ds(h * _CHUNK, _CHUNK)],
                        out_hbm.at[pl.ds(h * _N + base, _CHUNK)])


def _sc_gather(bb, idx_flat):
    mesh = plsc.VectorSubcoreMesh(core_axis_name="c", subcore_axis_name="s")
    return pl.kernel(
        _sc_gather_body,
        mesh=mesh,
        compiler_params=pltpu.CompilerParams(needs_layout_passes=False),
        out_type=jax.ShapeDtypeStruct((_H * _N,), jnp.float32),
        scratch_types=[
            pltpu.VMEM((_CHUNK,), jnp.int32),
            pltpu.VMEM((196 * _H,), jnp.float32),
            pltpu.VMEM((_H * _CHUNK,), jnp.float32),
        ],
    )(bb, idx_flat)


_G = 2401
_GBLK = 49
_NC = _G // _GBLK
_NBUF = 4


def _addp_body(x_hbm, pb_hbm, o_hbm, inb, outb, pbv, insem, outsem, pbsem):
    pltpu.make_async_copy(pb_hbm, pbv, pbsem).start()
    for i in range(_NBUF):
        pltpu.make_async_copy(x_hbm.at[pl.ds(i * _GBLK, _GBLK)], inb.at[i],
                              insem.at[i]).start()
    pltpu.make_async_copy(pb_hbm, pbv, pbsem).wait()

    row = jax.lax.broadcasted_iota(jnp.int32, (16, 2048), 0)
    col = jax.lax.broadcasted_iota(jnp.int32, (16, 2048), 1)
    rep = (row == col // 128).astype(jnp.float32)

    def chunk(c, carry):
        buf = lax.rem(c, _NBUF)
        pltpu.make_async_copy(x_hbm.at[pl.ds(c * _GBLK, _GBLK)], inb.at[buf],
                              insem.at[buf]).wait()

        @pl.when(c >= _NBUF)
        def _():
            pltpu.make_async_copy(
                outb.at[buf], o_hbm.at[pl.ds((c - _NBUF) * _GBLK, _GBLK)],
                outsem.at[buf]).wait()

        for g in range(_GBLK):
            t = jax.lax.dot_general(pbv[c * _GBLK + g], rep,
                                    (((1,), (0,)), ((), ())),
                                    preferred_element_type=jnp.float32)
            for j in range(16):
                outb[buf, g, j] = inb[buf, g, j] + t[:, j * 128:(j + 1) * 128]

        pltpu.make_async_copy(outb.at[buf], o_hbm.at[pl.ds(c * _GBLK, _GBLK)],
                              outsem.at[buf]).start()

        @pl.when(c + _NBUF < _NC)
        def _():
            pltpu.make_async_copy(x_hbm.at[pl.ds((c + _NBUF) * _GBLK, _GBLK)],
                                  inb.at[buf], insem.at[buf]).start()

        return carry

    lax.fori_loop(0, _NC, chunk, 0)
    for i in range(_NC - _NBUF, _NC):
        pltpu.make_async_copy(outb.at[i % _NBUF],
                              o_hbm.at[pl.ds(i * _GBLK, _GBLK)],
                              outsem.at[i % _NBUF]).wait()


def _tc_add(xt, pbt):
    return pl.pallas_call(
        _addp_body,
        in_specs=[
            pl.BlockSpec(memory_space=pl.ANY),
            pl.BlockSpec(memory_space=pl.ANY),
        ],
        out_specs=pl.BlockSpec(memory_space=pl.ANY),
        out_shape=jax.ShapeDtypeStruct((_G, 16, 8, 128), jnp.float32),
        scratch_shapes=[
            pltpu.VMEM((_NBUF, _GBLK, 16, 8, 128), jnp.float32),
            pltpu.VMEM((_NBUF, _GBLK, 16, 8, 128), jnp.float32),
            pltpu.VMEM((_G, 8, 16), jnp.float32),
            pltpu.SemaphoreType.DMA((_NBUF,)),
            pltpu.SemaphoreType.DMA((_NBUF,)),
            pltpu.SemaphoreType.DMA,
        ],
        compiler_params=pltpu.CompilerParams(vmem_limit_bytes=52 * 1024 * 1024),
    )(xt, pbt)


@jax.jit
def kernel(inputs, bb):
    B, H, QQ, KK = inputs.shape
    idx_flat = jnp.asarray(_bb_pos_flat(QQ, KK))
    pb = _sc_gather(bb.reshape(-1), idx_flat)
    xt = jnp.transpose(inputs, (2, 3, 1, 0)).reshape(_G, 16, H, B)
    pbt = jnp.transpose(pb.reshape(H, _G, 16), (1, 0, 2))
    ot = _tc_add(xt, pbt)
    return jnp.transpose(ot.reshape(QQ, KK, H, B), (3, 2, 0, 1))
